# baseline (speedup 1.0000x reference)
import functools

import jax
import jax.numpy as jnp
from jax import lax
from jax.experimental import pallas as pl
from jax.experimental.pallas import tpu as pltpu


def _round_up(x, m):
    return ((x + m - 1) // m) * m


def _row_dinv(a_block):
    s = jnp.sum(a_block, axis=1, keepdims=True, dtype=jnp.float32)
    return jnp.where(s > 0, lax.rsqrt(s), 0.0)


def _proj_kernel(x_ref, w_ref, o_ref):
    o_ref[...] = jnp.dot(
        x_ref[...].astype(jnp.bfloat16),
        w_ref[...],
        preferred_element_type=jnp.float32,
    ).astype(o_ref.dtype)


def _prep_kernel(a_ref, o_ref, d_ref):
    a = a_ref[...]
    o_ref[...] = a.astype(jnp.bfloat16)
    d_ref[...] = jnp.broadcast_to(_row_dinv(a), d_ref.shape)


def _layer1_kernel(a_ref, xw_ref, b1_ref, w2_ref, o_ref):
    a = a_ref[...]
    d = _row_dinv(a)
    acc = jnp.dot(a, xw_ref[...], preferred_element_type=jnp.float32)
    h = jnp.maximum(acc * d + b1_ref[...], 0.0).astype(jnp.bfloat16)
    hw = jnp.dot(h, w2_ref[...], preferred_element_type=jnp.float32)
    o_ref[...] = (hw * d).astype(o_ref.dtype)


def _layer2_kernel(num_classes, a_ref, hw_ref, b2_ref, o_ref):
    a = a_ref[...]
    d = _row_dinv(a)
    acc = jnp.dot(a, hw_ref[...], preferred_element_type=jnp.float32)
    z = acc * d + b2_ref[...]
    col = lax.broadcasted_iota(jnp.int32, z.shape, 1)
    valid = col < num_classes
    zm = jnp.where(valid, z, jnp.float32(-jnp.inf))
    m = jnp.max(zm, axis=1, keepdims=True)
    s = zm - m
    ssum = jnp.sum(jnp.where(valid, jnp.exp(s), 0.0), axis=1, keepdims=True)
    o_ref[...] = (s - jnp.log(ssum))[:, :num_classes]


def kernel(x, edge_index, w1, b1, w2, b2):
    n, fin = x.shape
    hidden = w1.shape[1]
    num_classes = w2.shape[1]

    np_ = max(128, _round_up(n, 128))
    tm = 512 if np_ % 512 == 0 else (256 if np_ % 256 == 0 else 128)
    hp = _round_up(max(hidden, 128), 128)
    cp = _round_up(max(num_classes, 128), 128)

    ar = jnp.arange(n, dtype=edge_index.dtype)
    rows = jnp.concatenate([edge_index[1], ar])
    cols = jnp.concatenate([edge_index[0], ar])
    ones = jnp.ones((rows.shape[0],), jnp.float32)
    a32 = jnp.zeros((np_, np_), jnp.float32).at[rows, cols].add(ones)

    a, dinv_b = pl.pallas_call(
        _prep_kernel,
        out_shape=(
            jax.ShapeDtypeStruct((np_, np_), jnp.bfloat16),
            jax.ShapeDtypeStruct((np_, 128), jnp.float32),
        ),
        grid=(np_ // tm,),
        in_specs=[pl.BlockSpec((tm, np_), lambda i: (i, 0))],
        out_specs=(
            pl.BlockSpec((tm, np_), lambda i: (i, 0)),
            pl.BlockSpec((tm, 128), lambda i: (i, 0)),
        ),
        compiler_params=pltpu.CompilerParams(
            dimension_semantics=("parallel",),
            vmem_limit_bytes=64 << 20,
        ),
        cost_estimate=pl.CostEstimate(
            flops=np_ * np_,
            transcendentals=np_,
            bytes_accessed=np_ * np_ * 4 + np_ * np_ * 2 + np_ * 128 * 4,
        ),
    )(a32)
    dinv = dinv_b[:, 0]

    if np_ > n:
        x = jnp.zeros((np_, fin), x.dtype).at[:n, :].set(x)
    w1p = jnp.zeros((fin, hp), jnp.bfloat16).at[:, :hidden].set(
        w1.astype(jnp.bfloat16))
    b1p = jnp.zeros((1, hp), jnp.float32).at[:, :hidden].set(
        b1.reshape(1, -1))
    w2p = jnp.zeros((hp, cp), jnp.bfloat16).at[:hidden, :num_classes].set(
        w2.astype(jnp.bfloat16))
    b2p = jnp.zeros((1, cp), jnp.float32).at[:, :num_classes].set(
        b2.reshape(1, -1))

    xw1 = pl.pallas_call(
        _proj_kernel,
        out_shape=jax.ShapeDtypeStruct((np_, hp), jnp.bfloat16),
        grid=(np_ // tm,),
        in_specs=[
            pl.BlockSpec((tm, fin), lambda i: (i, 0)),
            pl.BlockSpec((fin, hp), lambda i: (0, 0)),
        ],
        out_specs=pl.BlockSpec((tm, hp), lambda i: (i, 0)),
        compiler_params=pltpu.CompilerParams(
            dimension_semantics=("parallel",),
            vmem_limit_bytes=64 << 20,
        ),
        cost_estimate=pl.CostEstimate(
            flops=2 * np_ * fin * hp,
            transcendentals=0,
            bytes_accessed=np_ * fin * 4 + fin * hp * 2 + np_ * hp * 2,
        ),
    )(x, w1p)

    xw1s = (xw1.astype(jnp.float32) * dinv[:, None]).astype(jnp.bfloat16)

    hw2 = pl.pallas_call(
        _layer1_kernel,
        out_shape=jax.ShapeDtypeStruct((np_, cp), jnp.bfloat16),
        grid=(np_ // tm,),
        in_specs=[
            pl.BlockSpec((tm, np_), lambda i: (i, 0)),
            pl.BlockSpec((np_, hp), lambda i: (0, 0)),
            pl.BlockSpec((1, hp), lambda i: (0, 0)),
            pl.BlockSpec((hp, cp), lambda i: (0, 0)),
        ],
        out_specs=pl.BlockSpec((tm, cp), lambda i: (i, 0)),
        compiler_params=pltpu.CompilerParams(
            dimension_semantics=("parallel",),
            vmem_limit_bytes=64 << 20,
        ),
        cost_estimate=pl.CostEstimate(
            flops=2 * np_ * np_ * hp + 2 * np_ * hp * cp,
            transcendentals=0,
            bytes_accessed=np_ * np_ * 2 + np_ * hp * 2 + np_ * cp * 2,
        ),
    )(a, xw1s, b1p, w2p)

    out = pl.pallas_call(
        functools.partial(_layer2_kernel, num_classes),
        out_shape=jax.ShapeDtypeStruct((np_, num_classes), jnp.float32),
        grid=(np_ // tm,),
        in_specs=[
            pl.BlockSpec((tm, np_), lambda i: (i, 0)),
            pl.BlockSpec((np_, cp), lambda i: (0, 0)),
            pl.BlockSpec((1, cp), lambda i: (0, 0)),
        ],
        out_specs=pl.BlockSpec((tm, num_classes), lambda i: (i, 0)),
        compiler_params=pltpu.CompilerParams(
            dimension_semantics=("parallel",),
            vmem_limit_bytes=64 << 20,
        ),
        cost_estimate=pl.CostEstimate(
            flops=2 * np_ * np_ * cp,
            transcendentals=np_ * cp + np_,
            bytes_accessed=np_ * np_ * 2 + np_ * cp * 2 + np_ * num_classes * 4,
        ),
    )(a, hw2, b2p)

    return out[:n]


# speedup vs baseline: 1.0215x; 1.0215x over previous
import functools

import jax
import jax.numpy as jnp
from jax import lax
from jax.experimental import pallas as pl
from jax.experimental.pallas import tpu as pltpu


def _round_up(x, m):
    return ((x + m - 1) // m) * m


def _row_dinv(a_block):
    s = 1.0 + jnp.sum(a_block, axis=1, keepdims=True, dtype=jnp.float32)
    return lax.rsqrt(s)


def _proj_kernel(x_ref, w_ref, o_ref):
    o_ref[...] = jnp.dot(
        x_ref[...].astype(jnp.bfloat16),
        w_ref[...],
        preferred_element_type=jnp.float32,
    ).astype(o_ref.dtype)


def _layer1_kernel(tm, a_ref, xw_ref, b1_ref, w2_ref, o_ref):
    i = pl.program_id(0)
    a = a_ref[...]
    d = _row_dinv(a)
    acc = jnp.dot(a.astype(jnp.bfloat16), xw_ref[...],
                  preferred_element_type=jnp.float32)
    acc = acc + xw_ref[pl.ds(i * tm, tm), :].astype(jnp.float32)
    h = jnp.maximum(acc * d + b1_ref[...], 0.0).astype(jnp.bfloat16)
    hw = jnp.dot(h, w2_ref[...], preferred_element_type=jnp.float32)
    o_ref[...] = (hw * d).astype(o_ref.dtype)


def _layer2_kernel(tm, num_classes, a_ref, hw_ref, b2_ref, o_ref):
    i = pl.program_id(0)
    a = a_ref[...]
    d = _row_dinv(a)
    acc = jnp.dot(a.astype(jnp.bfloat16), hw_ref[...],
                  preferred_element_type=jnp.float32)
    acc = acc + hw_ref[pl.ds(i * tm, tm), :].astype(jnp.float32)
    z = acc * d + b2_ref[...]
    col = lax.broadcasted_iota(jnp.int32, z.shape, 1)
    valid = col < num_classes
    zm = jnp.where(valid, z, jnp.float32(-jnp.inf))
    m = jnp.max(zm, axis=1, keepdims=True)
    s = zm - m
    ssum = jnp.sum(jnp.where(valid, jnp.exp(s), 0.0), axis=1, keepdims=True)
    o_ref[...] = (s - jnp.log(ssum))[:, :num_classes]


def kernel(x, edge_index, w1, b1, w2, b2):
    n, fin = x.shape
    hidden = w1.shape[1]
    num_classes = w2.shape[1]

    np_ = max(128, _round_up(n, 128))
    tm = 512 if np_ % 512 == 0 else (256 if np_ % 256 == 0 else 128)
    hp = _round_up(max(hidden, 128), 128)
    cp = _round_up(max(num_classes, 128), 128)

    ones = jnp.ones((edge_index.shape[1],), jnp.float32)
    a = jnp.zeros((np_, np_), jnp.float32).at[
        edge_index[1], edge_index[0]].add(ones)

    deg = 1.0 + jnp.sum(a, axis=1, dtype=jnp.float32)
    dinv = lax.rsqrt(deg)

    if np_ > n:
        x = jnp.zeros((np_, fin), x.dtype).at[:n, :].set(x)
    w1p = jnp.zeros((fin, hp), jnp.bfloat16).at[:, :hidden].set(
        w1.astype(jnp.bfloat16))
    b1p = jnp.zeros((1, hp), jnp.float32).at[:, :hidden].set(
        b1.reshape(1, -1))
    w2p = jnp.zeros((hp, cp), jnp.bfloat16).at[:hidden, :num_classes].set(
        w2.astype(jnp.bfloat16))
    b2p = jnp.zeros((1, cp), jnp.float32).at[:, :num_classes].set(
        b2.reshape(1, -1))

    xw1 = pl.pallas_call(
        _proj_kernel,
        out_shape=jax.ShapeDtypeStruct((np_, hp), jnp.bfloat16),
        grid=(np_ // tm,),
        in_specs=[
            pl.BlockSpec((tm, fin), lambda i: (i, 0)),
            pl.BlockSpec((fin, hp), lambda i: (0, 0)),
        ],
        out_specs=pl.BlockSpec((tm, hp), lambda i: (i, 0)),
        compiler_params=pltpu.CompilerParams(
            dimension_semantics=("parallel",),
            vmem_limit_bytes=64 << 20,
        ),
        cost_estimate=pl.CostEstimate(
            flops=2 * np_ * fin * hp,
            transcendentals=0,
            bytes_accessed=np_ * fin * 4 + fin * hp * 2 + np_ * hp * 2,
        ),
    )(x, w1p)

    xw1s = (xw1.astype(jnp.float32) * dinv[:, None]).astype(jnp.bfloat16)

    hw2 = pl.pallas_call(
        functools.partial(_layer1_kernel, tm),
        out_shape=jax.ShapeDtypeStruct((np_, cp), jnp.bfloat16),
        grid=(np_ // tm,),
        in_specs=[
            pl.BlockSpec((tm, np_), lambda i: (i, 0)),
            pl.BlockSpec((np_, hp), lambda i: (0, 0)),
            pl.BlockSpec((1, hp), lambda i: (0, 0)),
            pl.BlockSpec((hp, cp), lambda i: (0, 0)),
        ],
        out_specs=pl.BlockSpec((tm, cp), lambda i: (i, 0)),
        compiler_params=pltpu.CompilerParams(
            dimension_semantics=("parallel",),
            vmem_limit_bytes=64 << 20,
        ),
        cost_estimate=pl.CostEstimate(
            flops=2 * np_ * np_ * hp + 2 * np_ * hp * cp,
            transcendentals=0,
            bytes_accessed=np_ * np_ * 4 + np_ * hp * 2 + np_ * cp * 2,
        ),
    )(a, xw1s, b1p, w2p)

    out = pl.pallas_call(
        functools.partial(_layer2_kernel, tm, num_classes),
        out_shape=jax.ShapeDtypeStruct((np_, num_classes), jnp.float32),
        grid=(np_ // tm,),
        in_specs=[
            pl.BlockSpec((tm, np_), lambda i: (i, 0)),
            pl.BlockSpec((np_, cp), lambda i: (0, 0)),
            pl.BlockSpec((1, cp), lambda i: (0, 0)),
        ],
        out_specs=pl.BlockSpec((tm, num_classes), lambda i: (i, 0)),
        compiler_params=pltpu.CompilerParams(
            dimension_semantics=("parallel",),
            vmem_limit_bytes=64 << 20,
        ),
        cost_estimate=pl.CostEstimate(
            flops=2 * np_ * np_ * cp,
            transcendentals=np_ * cp + np_,
            bytes_accessed=np_ * np_ * 4 + np_ * cp * 2 + np_ * num_classes * 4,
        ),
    )(a, hw2, b2p)

    return out[:n]
